# baseline (speedup 1.0000x reference)
import math

import jax
import jax.numpy as jnp
from jax import lax
from jax.experimental import pallas as pl
from jax.experimental.pallas import tpu as pltpu
from jax.experimental.pallas import tpu_sc as plsc

_VOCAB = 1000000
_EMB = 32
_SCALE = float(math.sqrt(_EMB))

_NC = 2
_NS = 16
_NW = _NC * _NS

_B = 16384 * 50
_C = 128
_CHUNKS = _B // (_C * _NW)


def _body(tok_hbm, table_hbm, out_hbm, idx_all, rows, gsem):
    wid = lax.axis_index("s") * _NC + lax.axis_index("c")
    cbase = wid * _CHUNKS
    pltpu.sync_copy(tok_hbm.at[pl.ds(cbase, _CHUNKS)], idx_all)

    def chunk(g, carry):
        pltpu.async_copy(table_hbm.at[idx_all.at[g]], rows, gsem).wait()

        def scale(i, c2):
            rows[i, pl.ds(0, 16)] = rows[i, pl.ds(0, 16)] * _SCALE
            rows[i, pl.ds(16, 16)] = rows[i, pl.ds(16, 16)] * _SCALE
            return c2

        lax.fori_loop(0, _C, scale, 0)
        row0 = (cbase + g) * _C
        pltpu.sync_copy(rows, out_hbm.at[pl.ds(row0, _C)])
        return carry

    lax.fori_loop(0, _CHUNKS, chunk, 0)


def kernel(tokens, table):
    tok = tokens.reshape(_B // _C, _C).astype(jnp.int32)
    mesh = plsc.VectorSubcoreMesh(
        core_axis_name="c", subcore_axis_name="s",
        num_cores=_NC, num_subcores=_NS,
    )
    out = pl.kernel(
        _body,
        out_type=jax.ShapeDtypeStruct((_B, _EMB), jnp.float32),
        mesh=mesh,
        compiler_params=pltpu.CompilerParams(use_tc_tiling_on_sc=False),
        scratch_types=[
            pltpu.VMEM((_CHUNKS, _C), jnp.int32),
            pltpu.VMEM((_C, _EMB), jnp.float32),
            pltpu.SemaphoreType.DMA,
        ],
    )(tok, table)
    return out.reshape(tokens.shape[0], tokens.shape[1], _EMB)


# speedup vs baseline: 1.1538x; 1.1538x over previous
import math

import jax
import jax.numpy as jnp
from jax import lax
from jax.experimental import pallas as pl
from jax.experimental.pallas import tpu as pltpu
from jax.experimental.pallas import tpu_sc as plsc

_EMB = 32
_SCALE = float(math.sqrt(_EMB))

_NC = 2
_NS = 16
_NW = _NC * _NS

_B = 16384 * 50
_C = 128
_CHUNKS = _B // (_C * _NW)
_NBUF = 4
_T = _CHUNKS // _NBUF


def _body(tok_hbm, table_hbm, out_hbm, idx_all,
          r0, r1, r2, r3, o0, o1, o2, o3,
          g0, g1, g2, g3, s0, s1, s2, s3):
    rows = (r0, r1, r2, r3)
    obuf = (o0, o1, o2, o3)
    gsem = (g0, g1, g2, g3)
    ssem = (s0, s1, s2, s3)

    wid = lax.axis_index("s") * _NC + lax.axis_index("c")
    cbase = wid * _CHUNKS
    pltpu.sync_copy(tok_hbm.at[pl.ds(cbase, _CHUNKS)], idx_all)

    for b in range(_NBUF):
        pltpu.async_copy(table_hbm.at[idx_all.at[b]], rows[b], gsem[b])

    def outer(t, carry):
        for b in range(_NBUF):
            g = t * _NBUF + b
            pltpu.make_async_copy(
                table_hbm.at[idx_all.at[b]], rows[b], gsem[b]).wait()

            @pl.when(t > 0)
            def _wait_store():
                pltpu.make_async_copy(
                    obuf[b], out_hbm.at[pl.ds(0, _C)], ssem[b]).wait()

            def scale(i, c2):
                base = i * 8
                for r in range(8):
                    obuf[b][base + r, pl.ds(0, 16)] = (
                        rows[b][base + r, pl.ds(0, 16)] * _SCALE)
                    obuf[b][base + r, pl.ds(16, 16)] = (
                        rows[b][base + r, pl.ds(16, 16)] * _SCALE)
                return c2

            lax.fori_loop(0, _C // 8, scale, 0)

            row0 = (cbase + g) * _C
            pltpu.async_copy(obuf[b], out_hbm.at[pl.ds(row0, _C)], ssem[b])

            @pl.when(g + _NBUF < _CHUNKS)
            def _next_gather():
                pltpu.async_copy(
                    table_hbm.at[idx_all.at[g + _NBUF]], rows[b], gsem[b])
        return carry

    lax.fori_loop(0, _T, outer, 0)

    for b in range(_NBUF):
        pltpu.make_async_copy(
            obuf[b], out_hbm.at[pl.ds(0, _C)], ssem[b]).wait()


def kernel(tokens, table):
    tok = tokens.reshape(_B // _C, _C).astype(jnp.int32)
    mesh = plsc.VectorSubcoreMesh(
        core_axis_name="c", subcore_axis_name="s",
        num_cores=_NC, num_subcores=_NS,
    )
    rowbuf = pltpu.VMEM((_C, _EMB), jnp.float32)
    out = pl.kernel(
        _body,
        out_type=jax.ShapeDtypeStruct((_B, _EMB), jnp.float32),
        mesh=mesh,
        compiler_params=pltpu.CompilerParams(use_tc_tiling_on_sc=False),
        scratch_types=(
            [pltpu.VMEM((_CHUNKS, _C), jnp.int32)]
            + [rowbuf] * (2 * _NBUF)
            + [pltpu.SemaphoreType.DMA] * (2 * _NBUF)
        ),
    )(tok, table)
    return out.reshape(tokens.shape[0], tokens.shape[1], _EMB)


# speedup vs baseline: 1.6090x; 1.3944x over previous
import math

import jax
import jax.numpy as jnp
from jax import lax
from jax.experimental import pallas as pl
from jax.experimental.pallas import tpu as pltpu
from jax.experimental.pallas import tpu_sc as plsc

_EMB = 32
_SCALE = float(math.sqrt(_EMB))

_NC = 2
_NS = 16
_NW = _NC * _NS

_BATCH = 16384
_SEQ = 50
_BT = _BATCH // 128
_BT_PER_W = _BT // _NW
_ROWS_PER_W = 128 * _BT_PER_W
_CHUNKS = _BT_PER_W * _SEQ
_NBUF = 4
_T = _CHUNKS // _NBUF


def _body(tok_hbm, table_hbm, out_hbm, idx_all,
          c0, c1, c2, c3, r0, r1, r2, r3, o0, o1, o2, o3,
          g0, g1, g2, g3, s0, s1, s2, s3):
    cbuf = (c0, c1, c2, c3)
    rows = (r0, r1, r2, r3)
    obuf = (o0, o1, o2, o3)
    gsem = (g0, g1, g2, g3)
    ssem = (s0, s1, s2, s3)

    wid = lax.axis_index("s") * _NC + lax.axis_index("c")
    pltpu.sync_copy(tok_hbm.at[pl.ds(wid * _ROWS_PER_W, _ROWS_PER_W)], idx_all)

    iota = jnp.arange(16, dtype=jnp.int32)

    def start_gather(g, b):
        btl = g // _SEQ
        s_ = g % _SEQ
        cols = jnp.full((16,), 0, jnp.int32) + s_
        for j in range(8):
            rids = btl * 128 + j * 16 + iota
            cbuf[b][pl.ds(j * 16, 16)] = plsc.load_gather(idx_all, [rids, cols])
        pltpu.async_copy(table_hbm.at[cbuf[b]], rows[b], gsem[b])

    for b in range(_NBUF):
        start_gather(b, b)

    def outer(t, carry):
        for b in range(_NBUF):
            g = t * _NBUF + b
            pltpu.make_async_copy(
                table_hbm.at[cbuf[b]], rows[b], gsem[b]).wait()

            @pl.when(t > 0)
            def _wait_store():
                for eg in range(4):
                    pltpu.make_async_copy(
                        obuf[b].at[eg], out_hbm.at[0, eg, 0], ssem[b]).wait()

            def tsc(e, c2):
                eg = e // 8
                e8 = e % 8
                eid = jnp.full((16,), 0, jnp.int32) + e
                for b16 in range(8):
                    bids = b16 * 16 + iota
                    v = plsc.load_gather(rows[b], [bids, eid])
                    obuf[b][eg, e8, pl.ds(b16 * 16, 16)] = v * _SCALE
                return c2

            lax.fori_loop(0, _EMB, tsc, 0)

            btl = g // _SEQ
            s_ = g % _SEQ
            btg = wid * _BT_PER_W + btl
            for eg in range(4):
                pltpu.async_copy(
                    obuf[b].at[eg], out_hbm.at[s_, eg, btg], ssem[b])

            @pl.when(g + _NBUF < _CHUNKS)
            def _next():
                start_gather(g + _NBUF, b)
        return carry

    lax.fori_loop(0, _T, outer, 0)

    for b in range(_NBUF):
        for eg in range(4):
            pltpu.make_async_copy(
                obuf[b].at[eg], out_hbm.at[0, eg, 0], ssem[b]).wait()


def kernel(tokens, table):
    mesh = plsc.VectorSubcoreMesh(
        core_axis_name="c", subcore_axis_name="s",
        num_cores=_NC, num_subcores=_NS,
    )
    out5 = pl.kernel(
        _body,
        out_type=jax.ShapeDtypeStruct((_SEQ, 4, _BT, 8, 128), jnp.float32),
        mesh=mesh,
        compiler_params=pltpu.CompilerParams(
            use_tc_tiling_on_sc=False, needs_layout_passes=False),
        scratch_types=(
            [pltpu.VMEM((_ROWS_PER_W, _SEQ), jnp.int32)]
            + [pltpu.VMEM((128,), jnp.int32)] * _NBUF
            + [pltpu.VMEM((128, _EMB), jnp.float32)] * _NBUF
            + [pltpu.VMEM((4, 8, 128), jnp.float32)] * _NBUF
            + [pltpu.SemaphoreType.DMA] * (2 * _NBUF)
        ),
    )(tokens, table)
    return out5.transpose(2, 4, 0, 1, 3).reshape(_BATCH, _SEQ, _EMB)


# speedup vs baseline: 2.4123x; 1.4993x over previous
import math

import jax
import jax.numpy as jnp
from jax import lax
from jax.experimental import pallas as pl
from jax.experimental.pallas import tpu as pltpu
from jax.experimental.pallas import tpu_sc as plsc

_EMB = 32
_SCALE = float(math.sqrt(_EMB))

_NC = 2
_NS = 16
_NW = _NC * _NS

_BATCH = 16384
_SEQ = 50
_BT = _BATCH // 128
_BT_PER_W = _BT // _NW
_ROWS_PER_W = 128 * _BT_PER_W
_CHUNKS = _BT_PER_W * _SEQ
_NBUF = 4
_T = _CHUNKS // _NBUF


def _body(tok_hbm, table_hbm, out_hbm, idx_all,
          r0, r1, r2, r3, o0, o1, o2, o3,
          g0, g1, g2, g3, s0, s1, s2, s3):
    rows = (r0, r1, r2, r3)
    obuf = (o0, o1, o2, o3)
    gsem = (g0, g1, g2, g3)
    ssem = (s0, s1, s2, s3)

    wid = lax.axis_index("s") * _NC + lax.axis_index("c")
    pltpu.sync_copy(
        tok_hbm.at[:, pl.ds(wid * _ROWS_PER_W, _ROWS_PER_W)], idx_all)

    iota = jnp.arange(16, dtype=jnp.int32)

    def idx_slice(g):
        btl = g // _SEQ
        s_ = g % _SEQ
        return idx_all.at[s_, pl.ds(btl * 128, 128)]

    def start_gather(g, b):
        pltpu.async_copy(table_hbm.at[idx_slice(g)], rows[b], gsem[b])

    for b in range(_NBUF):
        start_gather(b, b)

    def outer(t, carry):
        for b in range(_NBUF):
            g = t * _NBUF + b
            pltpu.make_async_copy(
                table_hbm.at[idx_slice(g)], rows[b], gsem[b]).wait()

            @pl.when(t > 0)
            def _wait_store():
                for eg in range(4):
                    pltpu.make_async_copy(
                        obuf[b].at[pl.ds(eg * 8, 8), pl.ds(0, 128)],
                        out_hbm.at[0, eg, 0], ssem[b]).wait()

            def tsc(r, c2):
                rid = jnp.full((16,), 0, jnp.int32) + r
                v0 = rows[b][r, pl.ds(0, 16)] * _SCALE
                plsc.store_scatter(obuf[b], [iota, rid], v0)
                v1 = rows[b][r, pl.ds(16, 16)] * _SCALE
                plsc.store_scatter(obuf[b], [iota + 16, rid], v1)
                return c2

            lax.fori_loop(0, 128, tsc, 0)

            btl = g // _SEQ
            s_ = g % _SEQ
            btg = wid * _BT_PER_W + btl
            for eg in range(4):
                pltpu.async_copy(
                    obuf[b].at[pl.ds(eg * 8, 8), pl.ds(0, 128)],
                    out_hbm.at[s_, eg, btg], ssem[b])

            @pl.when(g + _NBUF < _CHUNKS)
            def _next():
                start_gather(g + _NBUF, b)
        return carry

    lax.fori_loop(0, _T, outer, 0)

    for b in range(_NBUF):
        for eg in range(4):
            pltpu.make_async_copy(
                obuf[b].at[pl.ds(eg * 8, 8), pl.ds(0, 128)],
                out_hbm.at[0, eg, 0], ssem[b]).wait()


def kernel(tokens, table):
    tok_t = jnp.swapaxes(tokens, 0, 1)
    mesh = plsc.VectorSubcoreMesh(
        core_axis_name="c", subcore_axis_name="s",
        num_cores=_NC, num_subcores=_NS,
    )
    out5 = pl.kernel(
        _body,
        out_type=jax.ShapeDtypeStruct((_SEQ, 4, _BT, 8, 128), jnp.float32),
        mesh=mesh,
        compiler_params=pltpu.CompilerParams(
            use_tc_tiling_on_sc=False, needs_layout_passes=False),
        scratch_types=(
            [pltpu.VMEM((_SEQ, _ROWS_PER_W), jnp.int32)]
            + [pltpu.VMEM((128, _EMB), jnp.float32)] * _NBUF
            + [pltpu.VMEM((_EMB, 144), jnp.float32)] * _NBUF
            + [pltpu.SemaphoreType.DMA] * (2 * _NBUF)
        ),
    )(tok_t, table)
    return out5.transpose(2, 4, 0, 1, 3).reshape(_BATCH, _SEQ, _EMB)
